# Initial kernel scaffold; baseline (speedup 1.0000x reference)
import jax
import jax.numpy as jnp
from jax.experimental import pallas as pl


def kernel(x, edge_index, params):
    raise NotImplementedError("write your pallas kernel here")


# speedup vs baseline: 1.2295x; 1.2295x over previous
import functools

import jax
import jax.numpy as jnp
from jax import lax
from jax.experimental import pallas as pl
from jax.experimental.pallas import tpu as pltpu
from jax.experimental.pallas import tpu_sc as plsc

N = 10000
E = 320000
DH = 64
PCH = 128
NCHP = 1280
PAIRS = NCHP * PCH
EP = 2 * PAIRS
NW = 32
CPW = NCHP // NW
NACC = 10112
RPT = NACC // 16
EBLK = 1024
LN_EPS = 1e-6

_MESH = plsc.VectorSubcoreMesh(core_axis_name="c", subcore_axis_name="s")


@functools.partial(
    pl.kernel,
    out_type=jax.ShapeDtypeStruct((NCHP, PCH, 128), jnp.float32),
    mesh=_MESH,
    scratch_types=[
        pltpu.VMEM((PCH,), jnp.int32),
        pltpu.VMEM((PCH,), jnp.int32),
        pltpu.VMEM((PCH,), jnp.int32),
        pltpu.VMEM((PCH,), jnp.int32),
        pltpu.VMEM((PCH, 128), jnp.float32),
        pltpu.SemaphoreType.DMA,
    ],
)
def _sc_gather(tsa, tsb, tra, trb, se_h, so_h, re_h, ro_h, out,
               i1, i2, i3, i4, rows, sem):
    wid = lax.axis_index("s") * 2 + lax.axis_index("c")

    def body(t, carry):
        cb = wid * CPW + t
        pltpu.sync_copy(se_h.at[cb], i1)
        pltpu.sync_copy(so_h.at[cb], i2)
        pltpu.sync_copy(re_h.at[cb], i3)
        pltpu.sync_copy(ro_h.at[cb], i4)
        pltpu.async_copy(tsa.at[i1], rows, sem).wait()
        pltpu.sync_copy(tsb.at[i2], rows, add=True)
        pltpu.sync_copy(tra.at[i3], rows, add=True)
        pltpu.sync_copy(trb.at[i4], rows, add=True)
        pltpu.sync_copy(rows, out.at[cb])
        return carry

    lax.fori_loop(0, CPW, body, 0)


@functools.partial(
    pl.kernel,
    out_type=jax.ShapeDtypeStruct((2 * NACC, 128), jnp.float32),
    mesh=_MESH,
    scratch_types=[
        pltpu.VMEM((2, PCH), jnp.int32),
        pltpu.VMEM((PCH, 128), jnp.float32),
        pltpu.VMEM((PCH, 128), jnp.float32),
        pltpu.VMEM_SHARED((NACC, 128), jnp.float32),
    ],
)
def _sc_scatter(raw_e, raw_o, rse_h, rso_h, zero_h, out,
                idx2, rows_e, rows_o, acc):
    cid = lax.axis_index("c")
    sid = lax.axis_index("s")
    wid = sid * 2 + cid
    pltpu.sync_copy(zero_h.at[pl.ds(sid * RPT, RPT)],
                    acc.at[pl.ds(sid * RPT, RPT)])
    plsc.subcore_barrier()

    def body(t, carry):
        cb = wid * CPW + t
        pltpu.sync_copy(rse_h.at[cb], idx2.at[0])
        pltpu.sync_copy(rso_h.at[cb], idx2.at[1])
        pltpu.sync_copy(raw_e.at[cb], rows_e)
        pltpu.sync_copy(raw_o.at[cb], rows_o)
        pltpu.sync_copy(rows_e, acc.at[idx2.at[0]], add=True)
        pltpu.sync_copy(rows_o, acc.at[idx2.at[1]], add=True)
        return carry

    lax.fori_loop(0, CPW, body, 0)
    plsc.subcore_barrier()
    pltpu.sync_copy(acc.at[pl.ds(sid * RPT, RPT)],
                    out.at[pl.ds(cid * NACC + sid * RPT, RPT)])


def _full(shape):
    return pl.BlockSpec(shape, lambda *_: tuple(0 for _ in shape))


def _ln(x, scale, bias):
    mu = jnp.mean(x, axis=-1, keepdims=True)
    var = jnp.mean((x - mu) ** 2, axis=-1, keepdims=True)
    return (x - mu) / jnp.sqrt(var + LN_EPS) * scale + bias


def _enc_body(x_ref, w_ref, b_ref, wsa_ref, wsb_ref, wra_ref, wrb_ref,
              nodes_ref, tsa_ref, tsb_ref, tra_ref, trb_ref):
    n = jnp.dot(x_ref[...], w_ref[...]) + b_ref[...]
    nodes_ref[...] = n
    tsa_ref[...] = jnp.dot(n, wsa_ref[...])
    tsb_ref[...] = jnp.dot(n, wsb_ref[...])
    tra_ref[...] = jnp.dot(n, wra_ref[...])
    trb_ref[...] = jnp.dot(n, wrb_ref[...])


def _encode(x, w, b, wsa, wsb, wra, wrb):
    t128 = jax.ShapeDtypeStruct((N, 128), jnp.float32)
    return pl.pallas_call(
        _enc_body,
        out_shape=(jax.ShapeDtypeStruct((N, DH), jnp.float32),
                   t128, t128, t128, t128),
    )(x, w, b, wsa, wsb, wra, wrb)


def _edge_body(has_prev, want_ln, *refs):
    if has_prev:
        (pre_ref, eln_ref, w1e_ref, b1_ref, w2_ref, b2_ref,
         w3_ref, b3_ref, lns_ref, lnb_ref, *outs) = refs
    else:
        (pre_ref, b1_ref, w2_ref, b2_ref,
         w3_ref, b3_ref, lns_ref, lnb_ref, *outs) = refs
    x = pre_ref[...] + b1_ref[...]
    if has_prev:
        x = x + jnp.dot(eln_ref[...], w1e_ref[...])
    h = jnp.dot(jax.nn.gelu(x), w2_ref[...]) + b2_ref[...]
    h = jnp.dot(jax.nn.gelu(h), w3_ref[...]) + b3_ref[...]
    z = jnp.zeros_like(h[:, :DH])
    outs[0][...] = jnp.concatenate([h[:, :DH], z], axis=1)
    outs[1][...] = jnp.concatenate([h[:, DH:], z], axis=1)
    if want_ln:
        lne = _ln(h[:, :DH], lns_ref[...], lnb_ref[...])
        lno = _ln(h[:, DH:], lns_ref[...], lnb_ref[...])
        outs[2][...] = jnp.concatenate([lne, lno], axis=1)


def _edge_step(has_prev, want_ln, pre2, eln2, w1e_bd, b1_bd, w2_bd, b2_bd,
               w3_bd, b3_bd, lns, lnb):
    blk128 = lambda: pl.BlockSpec((EBLK, 128), lambda i: (i, 0))
    in_specs = [blk128()]
    args = [pre2]
    if has_prev:
        in_specs += [blk128(), _full((128, 128))]
        args += [eln2, w1e_bd]
    in_specs += [_full((1, 128)), _full((128, 128)), _full((1, 128)),
                 _full((128, 128)), _full((1, 128)),
                 _full((1, DH)), _full((1, DH))]
    args += [b1_bd, w2_bd, b2_bd, w3_bd, b3_bd, lns, lnb]
    out_specs = [blk128(), blk128()]
    out_shape = [jax.ShapeDtypeStruct((PAIRS, 128), jnp.float32),
                 jax.ShapeDtypeStruct((PAIRS, 128), jnp.float32)]
    if want_ln:
        out_specs.append(blk128())
        out_shape.append(jax.ShapeDtypeStruct((PAIRS, 128), jnp.float32))
    res = pl.pallas_call(
        functools.partial(_edge_body, has_prev, want_ln),
        grid=(PAIRS // EBLK,),
        in_specs=in_specs,
        out_specs=tuple(out_specs),
        out_shape=tuple(out_shape),
    )(*args)
    return res if want_ln else (res[0], res[1], None)


def _node_body(last, *refs):
    (nodes_ref, rec_ref, w1a_ref, w1b_ref, b1_ref, w2_ref, b2_ref,
     w3_ref, b3_ref, lns_ref, lnb_ref, *rest) = refs
    rec = (rec_ref[pl.ds(0, N), :] + rec_ref[pl.ds(NACC, N), :])[:, :DH]
    pre = (jnp.dot(nodes_ref[...], w1a_ref[...])
           + jnp.dot(rec, w1b_ref[...]) + b1_ref[...])
    h = jnp.dot(jax.nn.gelu(pre), w2_ref[...]) + b2_ref[...]
    h = jnp.dot(jax.nn.gelu(h), w3_ref[...]) + b3_ref[...]
    nodes_ln = _ln(h, lns_ref[...], lnb_ref[...])
    if not last:
        (wsa_ref, wsb_ref, wra_ref, wrb_ref,
         nodes_out, tsa_out, tsb_out, tra_out, trb_out) = rest
        nodes_out[...] = nodes_ln
        tsa_out[...] = jnp.dot(nodes_ln, wsa_ref[...])
        tsb_out[...] = jnp.dot(nodes_ln, wsb_ref[...])
        tra_out[...] = jnp.dot(nodes_ln, wra_ref[...])
        trb_out[...] = jnp.dot(nodes_ln, wrb_ref[...])
    else:
        (rw1_ref, rb1_ref, rw2_ref, rb2_ref, rw3_ref, rb3_ref,
         rw4_ref, rb4_ref, out_ref) = rest
        m = jnp.mean(nodes_ln, axis=0, keepdims=True)
        h = jax.nn.gelu(jnp.dot(m, rw1_ref[...]) + rb1_ref[...])
        h = jax.nn.gelu(jnp.dot(h, rw2_ref[...]) + rb2_ref[...])
        h = jax.nn.gelu(jnp.dot(h, rw3_ref[...]) + rb3_ref[...])
        out_ref[...] = jnp.dot(h, rw4_ref[...]) + rb4_ref[...]


def _node_step(nargs, wsa, wsb, wra, wrb):
    t128 = jax.ShapeDtypeStruct((N, 128), jnp.float32)
    return pl.pallas_call(
        functools.partial(_node_body, False),
        out_shape=(jax.ShapeDtypeStruct((N, DH), jnp.float32),
                   t128, t128, t128, t128),
    )(*nargs, wsa, wsb, wra, wrb)


def _node_readout(nargs, ro_args):
    return pl.pallas_call(
        functools.partial(_node_body, True),
        out_shape=jax.ShapeDtypeStruct((1, 1), jnp.float32),
    )(*nargs, *ro_args)


def _bd(w):
    z = jnp.zeros_like(w)
    return jnp.concatenate(
        [jnp.concatenate([w, z], axis=1), jnp.concatenate([z, w], axis=1)],
        axis=0)


def _padA(w):
    return jnp.concatenate([w, jnp.zeros_like(w)], axis=1)


def _padB(w):
    return jnp.concatenate([jnp.zeros_like(w), w], axis=1)


def kernel(x, edge_index, params):
    senders = edge_index[0].astype(jnp.int32)
    receivers = edge_index[1].astype(jnp.int32)
    pad = EP - E
    s_pad = jnp.concatenate([senders, jnp.zeros((pad,), jnp.int32)])
    r_pad = jnp.concatenate([receivers, jnp.zeros((pad,), jnp.int32)])
    rs_pad = jnp.concatenate([receivers, jnp.full((pad,), N, jnp.int32)])
    se = s_pad[0::2].reshape(NCHP, PCH)
    so = s_pad[1::2].reshape(NCHP, PCH)
    re_ = r_pad[0::2].reshape(NCHP, PCH)
    ro = r_pad[1::2].reshape(NCHP, PCH)
    rse = rs_pad[0::2].reshape(NCHP, PCH)
    rso = rs_pad[1::2].reshape(NCHP, PCH)
    zero_acc = jnp.zeros((NACC, 128), jnp.float32)

    def row(v):
        return v.reshape(1, -1)

    def row2(v):
        return jnp.concatenate([v, v]).reshape(1, -1)

    steps = params["steps"]
    w1_0 = steps[0]["edge_mlp"][0]["W"]
    nodes, tsa, tsb, tra, trb = _encode(
        x, params["enc"]["W"], row(params["enc"]["b"]),
        _padA(w1_0[:DH]), _padB(w1_0[:DH]),
        _padA(w1_0[DH:2 * DH]), _padB(w1_0[DH:2 * DH]))

    eln2 = None
    for t, sp in enumerate(steps):
        em, nm = sp["edge_mlp"], sp["node_mlp"]
        pre2 = _sc_gather(tsa, tsb, tra, trb, se, so, re_, ro)
        pre2 = pre2.reshape(PAIRS, 128)
        has_prev = t > 0
        want_ln = t < 2
        raw_e, raw_o, eln2 = _edge_step(
            has_prev, want_ln, pre2, eln2,
            _bd(em[0]["W"][:DH]) if has_prev else None,
            row2(em[0]["b"]), _bd(em[1]["W"]), row2(em[1]["b"]),
            _bd(em[2]["W"]), row2(em[2]["b"]),
            row(sp["ln_edges"]["scale"]), row(sp["ln_edges"]["bias"]))
        rec = _sc_scatter(raw_e.reshape(NCHP, PCH, 128),
                          raw_o.reshape(NCHP, PCH, 128), rse, rso, zero_acc)
        nw1 = nm[0]["W"]
        nargs = (nodes, rec, nw1[:DH], nw1[DH:], row(nm[0]["b"]),
                 nm[1]["W"], row(nm[1]["b"]), nm[2]["W"], row(nm[2]["b"]),
                 row(sp["ln_nodes"]["scale"]), row(sp["ln_nodes"]["bias"]))
        if t < 2:
            wn = steps[t + 1]["edge_mlp"][0]["W"]
            nodes, tsa, tsb, tra, trb = _node_step(
                nargs, _padA(wn[DH:2 * DH]), _padB(wn[DH:2 * DH]),
                _padA(wn[2 * DH:]), _padB(wn[2 * DH:]))
        else:
            ro_p = params["readout"]
            ro_args = (ro_p[0]["W"], row(ro_p[0]["b"]),
                       ro_p[1]["W"], row(ro_p[1]["b"]),
                       ro_p[2]["W"], row(ro_p[2]["b"]),
                       ro_p[3]["W"], row(ro_p[3]["b"]))
            out = _node_readout(nargs, ro_args)
    return out[0]


# speedup vs baseline: 1.3346x; 1.0855x over previous
import functools

import jax
import jax.numpy as jnp
from jax import lax
from jax.experimental import pallas as pl
from jax.experimental.pallas import tpu as pltpu
from jax.experimental.pallas import tpu_sc as plsc

N = 10000
E = 320000
DH = 64
PCH = 128
NCHP = 1280
PAIRS = NCHP * PCH
EP = 2 * PAIRS
NW = 32
CPW = NCHP // NW
GP = 512
NCHG = PAIRS // GP
GCPW = NCHG // NW
NACC = 10112
RPT = NACC // 16
EBLK = 1024
LN_EPS = 1e-6

_MESH = plsc.VectorSubcoreMesh(core_axis_name="c", subcore_axis_name="s")


@functools.partial(
    pl.kernel,
    out_type=jax.ShapeDtypeStruct((NCHG, GP, 128), jnp.float32),
    mesh=_MESH,
    scratch_types=[
        pltpu.VMEM((GP,), jnp.int32),
        pltpu.VMEM((GP,), jnp.int32),
        pltpu.VMEM((GP,), jnp.int32),
        pltpu.VMEM((GP,), jnp.int32),
        pltpu.VMEM((GP, 128), jnp.float32),
        pltpu.SemaphoreType.DMA,
    ],
)
def _sc_gather(tsa, tsb, tra, trb, idx4_h, out, i1, i2, i3, i4, rows, sem):
    wid = lax.axis_index("s") * 2 + lax.axis_index("c")

    def body(t, carry):
        cb = wid * GCPW + t
        pltpu.sync_copy(idx4_h.at[cb, 0], i1)
        pltpu.sync_copy(idx4_h.at[cb, 1], i2)
        pltpu.sync_copy(idx4_h.at[cb, 2], i3)
        pltpu.sync_copy(idx4_h.at[cb, 3], i4)
        pltpu.async_copy(tsa.at[i1], rows, sem).wait()
        pltpu.sync_copy(tsb.at[i2], rows, add=True)
        pltpu.sync_copy(tra.at[i3], rows, add=True)
        pltpu.sync_copy(trb.at[i4], rows, add=True)
        pltpu.sync_copy(rows, out.at[cb])
        return carry

    lax.fori_loop(0, GCPW, body, 0)


@functools.partial(
    pl.kernel,
    out_type=jax.ShapeDtypeStruct((2 * NACC, 128), jnp.float32),
    mesh=_MESH,
    scratch_types=[
        pltpu.VMEM((2, PCH), jnp.int32),
        pltpu.VMEM((PCH, 128), jnp.float32),
        pltpu.VMEM((PCH, 128), jnp.float32),
        pltpu.VMEM_SHARED((NACC, 128), jnp.float32),
    ],
)
def _sc_scatter(raw_e, raw_o, rse_h, rso_h, zero_h, out,
                idx2, rows_e, rows_o, acc):
    cid = lax.axis_index("c")
    sid = lax.axis_index("s")
    wid = sid * 2 + cid
    pltpu.sync_copy(zero_h.at[pl.ds(sid * RPT, RPT)],
                    acc.at[pl.ds(sid * RPT, RPT)])
    plsc.subcore_barrier()

    def body(t, carry):
        cb = wid * CPW + t
        pltpu.sync_copy(rse_h.at[cb], idx2.at[0])
        pltpu.sync_copy(rso_h.at[cb], idx2.at[1])
        pltpu.sync_copy(raw_e.at[cb], rows_e)
        pltpu.sync_copy(raw_o.at[cb], rows_o)
        pltpu.sync_copy(rows_e, acc.at[idx2.at[0]], add=True)
        pltpu.sync_copy(rows_o, acc.at[idx2.at[1]], add=True)
        return carry

    lax.fori_loop(0, CPW, body, 0)
    plsc.subcore_barrier()
    pltpu.sync_copy(acc.at[pl.ds(sid * RPT, RPT)],
                    out.at[pl.ds(cid * NACC + sid * RPT, RPT)])


def _full(shape):
    return pl.BlockSpec(shape, lambda *_: tuple(0 for _ in shape))


def _ln(x, scale, bias):
    mu = jnp.mean(x, axis=-1, keepdims=True)
    var = jnp.mean((x - mu) ** 2, axis=-1, keepdims=True)
    return (x - mu) / jnp.sqrt(var + LN_EPS) * scale + bias


def _enc_body(x_ref, w_ref, b_ref, wsa_ref, wsb_ref, wra_ref, wrb_ref,
              nodes_ref, tsa_ref, tsb_ref, tra_ref, trb_ref):
    n = jnp.dot(x_ref[...], w_ref[...]) + b_ref[...]
    nodes_ref[...] = n
    tsa_ref[...] = jnp.dot(n, wsa_ref[...])
    tsb_ref[...] = jnp.dot(n, wsb_ref[...])
    tra_ref[...] = jnp.dot(n, wra_ref[...])
    trb_ref[...] = jnp.dot(n, wrb_ref[...])


def _encode(x, w, b, wsa, wsb, wra, wrb):
    t128 = jax.ShapeDtypeStruct((N, 128), jnp.float32)
    return pl.pallas_call(
        _enc_body,
        out_shape=(jax.ShapeDtypeStruct((N, DH), jnp.float32),
                   t128, t128, t128, t128),
    )(x, w, b, wsa, wsb, wra, wrb)


def _edge_body(has_prev, want_ln, *refs):
    if has_prev:
        (pre_ref, eln_ref, w1e_ref, b1_ref, w2_ref, b2_ref,
         w3_ref, b3_ref, lns_ref, lnb_ref, *outs) = refs
    else:
        (pre_ref, b1_ref, w2_ref, b2_ref,
         w3_ref, b3_ref, lns_ref, lnb_ref, *outs) = refs
    x = pre_ref[...] + b1_ref[...]
    if has_prev:
        x = x + jnp.dot(eln_ref[...], w1e_ref[...])
    h = jnp.dot(jax.nn.gelu(x), w2_ref[...]) + b2_ref[...]
    h = jnp.dot(jax.nn.gelu(h), w3_ref[...]) + b3_ref[...]
    z = jnp.zeros_like(h[:, :DH])
    outs[0][...] = jnp.concatenate([h[:, :DH], z], axis=1)
    outs[1][...] = jnp.concatenate([h[:, DH:], z], axis=1)
    if want_ln:
        lne = _ln(h[:, :DH], lns_ref[...], lnb_ref[...])
        lno = _ln(h[:, DH:], lns_ref[...], lnb_ref[...])
        outs[2][...] = jnp.concatenate([lne, lno], axis=1)


def _edge_step(has_prev, want_ln, pre2, eln2, w1e_bd, b1_bd, w2_bd, b2_bd,
               w3_bd, b3_bd, lns, lnb):
    blk128 = lambda: pl.BlockSpec((EBLK, 128), lambda i: (i, 0))
    in_specs = [blk128()]
    args = [pre2]
    if has_prev:
        in_specs += [blk128(), _full((128, 128))]
        args += [eln2, w1e_bd]
    in_specs += [_full((1, 128)), _full((128, 128)), _full((1, 128)),
                 _full((128, 128)), _full((1, 128)),
                 _full((1, DH)), _full((1, DH))]
    args += [b1_bd, w2_bd, b2_bd, w3_bd, b3_bd, lns, lnb]
    out_specs = [blk128(), blk128()]
    out_shape = [jax.ShapeDtypeStruct((PAIRS, 128), jnp.float32),
                 jax.ShapeDtypeStruct((PAIRS, 128), jnp.float32)]
    if want_ln:
        out_specs.append(blk128())
        out_shape.append(jax.ShapeDtypeStruct((PAIRS, 128), jnp.float32))
    res = pl.pallas_call(
        functools.partial(_edge_body, has_prev, want_ln),
        grid=(PAIRS // EBLK,),
        in_specs=in_specs,
        out_specs=tuple(out_specs),
        out_shape=tuple(out_shape),
    )(*args)
    return res if want_ln else (res[0], res[1], None)


def _node_body(last, *refs):
    (nodes_ref, rec_ref, w1a_ref, w1b_ref, b1_ref, w2_ref, b2_ref,
     w3_ref, b3_ref, lns_ref, lnb_ref, *rest) = refs
    rec = (rec_ref[pl.ds(0, N), :] + rec_ref[pl.ds(NACC, N), :])[:, :DH]
    pre = (jnp.dot(nodes_ref[...], w1a_ref[...])
           + jnp.dot(rec, w1b_ref[...]) + b1_ref[...])
    h = jnp.dot(jax.nn.gelu(pre), w2_ref[...]) + b2_ref[...]
    h = jnp.dot(jax.nn.gelu(h), w3_ref[...]) + b3_ref[...]
    nodes_ln = _ln(h, lns_ref[...], lnb_ref[...])
    if not last:
        (wsa_ref, wsb_ref, wra_ref, wrb_ref,
         nodes_out, tsa_out, tsb_out, tra_out, trb_out) = rest
        nodes_out[...] = nodes_ln
        tsa_out[...] = jnp.dot(nodes_ln, wsa_ref[...])
        tsb_out[...] = jnp.dot(nodes_ln, wsb_ref[...])
        tra_out[...] = jnp.dot(nodes_ln, wra_ref[...])
        trb_out[...] = jnp.dot(nodes_ln, wrb_ref[...])
    else:
        (rw1_ref, rb1_ref, rw2_ref, rb2_ref, rw3_ref, rb3_ref,
         rw4_ref, rb4_ref, out_ref) = rest
        m = jnp.mean(nodes_ln, axis=0, keepdims=True)
        h = jax.nn.gelu(jnp.dot(m, rw1_ref[...]) + rb1_ref[...])
        h = jax.nn.gelu(jnp.dot(h, rw2_ref[...]) + rb2_ref[...])
        h = jax.nn.gelu(jnp.dot(h, rw3_ref[...]) + rb3_ref[...])
        out_ref[...] = jnp.dot(h, rw4_ref[...]) + rb4_ref[...]


def _node_step(nargs, wsa, wsb, wra, wrb):
    t128 = jax.ShapeDtypeStruct((N, 128), jnp.float32)
    return pl.pallas_call(
        functools.partial(_node_body, False),
        out_shape=(jax.ShapeDtypeStruct((N, DH), jnp.float32),
                   t128, t128, t128, t128),
    )(*nargs, wsa, wsb, wra, wrb)


def _node_readout(nargs, ro_args):
    return pl.pallas_call(
        functools.partial(_node_body, True),
        out_shape=jax.ShapeDtypeStruct((1, 1), jnp.float32),
    )(*nargs, *ro_args)


def _bd(w):
    z = jnp.zeros_like(w)
    return jnp.concatenate(
        [jnp.concatenate([w, z], axis=1), jnp.concatenate([z, w], axis=1)],
        axis=0)


def _padA(w):
    return jnp.concatenate([w, jnp.zeros_like(w)], axis=1)


def _padB(w):
    return jnp.concatenate([jnp.zeros_like(w), w], axis=1)


def kernel(x, edge_index, params):
    senders = edge_index[0].astype(jnp.int32)
    receivers = edge_index[1].astype(jnp.int32)
    pad = EP - E
    s_pad = jnp.concatenate([senders, jnp.zeros((pad,), jnp.int32)])
    r_pad = jnp.concatenate([receivers, jnp.zeros((pad,), jnp.int32)])
    rs_pad = jnp.concatenate([receivers, jnp.full((pad,), N, jnp.int32)])
    idx4 = jnp.stack([s_pad[0::2].reshape(NCHG, GP),
                      s_pad[1::2].reshape(NCHG, GP),
                      r_pad[0::2].reshape(NCHG, GP),
                      r_pad[1::2].reshape(NCHG, GP)], axis=1)
    rse = rs_pad[0::2].reshape(NCHP, PCH)
    rso = rs_pad[1::2].reshape(NCHP, PCH)
    zero_acc = jnp.zeros((NACC, 128), jnp.float32)

    def row(v):
        return v.reshape(1, -1)

    def row2(v):
        return jnp.concatenate([v, v]).reshape(1, -1)

    steps = params["steps"]
    w1_0 = steps[0]["edge_mlp"][0]["W"]
    nodes, tsa, tsb, tra, trb = _encode(
        x, params["enc"]["W"], row(params["enc"]["b"]),
        _padA(w1_0[:DH]), _padB(w1_0[:DH]),
        _padA(w1_0[DH:2 * DH]), _padB(w1_0[DH:2 * DH]))

    eln2 = None
    for t, sp in enumerate(steps):
        em, nm = sp["edge_mlp"], sp["node_mlp"]
        pre2 = _sc_gather(tsa, tsb, tra, trb, idx4)
        pre2 = pre2.reshape(PAIRS, 128)
        has_prev = t > 0
        want_ln = t < 2
        raw_e, raw_o, eln2 = _edge_step(
            has_prev, want_ln, pre2, eln2,
            _bd(em[0]["W"][:DH]) if has_prev else None,
            row2(em[0]["b"]), _bd(em[1]["W"]), row2(em[1]["b"]),
            _bd(em[2]["W"]), row2(em[2]["b"]),
            row(sp["ln_edges"]["scale"]), row(sp["ln_edges"]["bias"]))
        rec = _sc_scatter(raw_e.reshape(NCHP, PCH, 128),
                          raw_o.reshape(NCHP, PCH, 128), rse, rso, zero_acc)
        nw1 = nm[0]["W"]
        nargs = (nodes, rec, nw1[:DH], nw1[DH:], row(nm[0]["b"]),
                 nm[1]["W"], row(nm[1]["b"]), nm[2]["W"], row(nm[2]["b"]),
                 row(sp["ln_nodes"]["scale"]), row(sp["ln_nodes"]["bias"]))
        if t < 2:
            wn = steps[t + 1]["edge_mlp"][0]["W"]
            nodes, tsa, tsb, tra, trb = _node_step(
                nargs, _padA(wn[DH:2 * DH]), _padB(wn[DH:2 * DH]),
                _padA(wn[2 * DH:]), _padB(wn[2 * DH:]))
        else:
            ro_p = params["readout"]
            ro_args = (ro_p[0]["W"], row(ro_p[0]["b"]),
                       ro_p[1]["W"], row(ro_p[1]["b"]),
                       ro_p[2]["W"], row(ro_p[2]["b"]),
                       ro_p[3]["W"], row(ro_p[3]["b"]))
            out = _node_readout(nargs, ro_args)
    return out[0]


# speedup vs baseline: 1.4656x; 1.0981x over previous
import functools

import jax
import jax.numpy as jnp
from jax import lax
from jax.experimental import pallas as pl
from jax.experimental.pallas import tpu as pltpu
from jax.experimental.pallas import tpu_sc as plsc

N = 10000
E = 320000
DH = 64
PCH = 128
NCHP = 1280
PAIRS = NCHP * PCH
EP = 2 * PAIRS
NW = 32
CPW = NCHP // NW
GP = 320
NCHG = PAIRS // GP
GCPW = NCHG // NW
NACC = 10112
RPT = NACC // 16
EBLK = 1024
LN_EPS = 1e-6

_MESH = plsc.VectorSubcoreMesh(core_axis_name="c", subcore_axis_name="s")


@functools.partial(
    pl.kernel,
    out_type=jax.ShapeDtypeStruct((NCHG, GP, 128), jnp.float32),
    mesh=_MESH,
    scratch_types=[
        pltpu.VMEM((4 * GP,), jnp.int32),
        pltpu.VMEM((4 * GP,), jnp.int32),
        pltpu.VMEM((GP, 128), jnp.float32),
        pltpu.VMEM((GP, 128), jnp.float32),
        pltpu.SemaphoreType.DMA,
        pltpu.SemaphoreType.DMA,
    ],
)
def _sc_gather(tsa, tsb, tra, trb, idx4_h, out,
               idx0, idx1, rows0, rows1, sem0, sem1):
    wid = lax.axis_index("s") * 2 + lax.axis_index("c")
    base = wid * GCPW
    tables = (tsa, tsb, tra, trb)
    for pr in range(GCPW // 2):
        c0 = base + 2 * pr
        c1 = c0 + 1
        pltpu.sync_copy(idx4_h.at[c0], idx0)
        h0 = pltpu.async_copy(tables[0].at[idx0.at[pl.ds(0, GP)]], rows0, sem0)
        pltpu.sync_copy(idx4_h.at[c1], idx1)
        h1 = pltpu.async_copy(tables[0].at[idx1.at[pl.ds(0, GP)]], rows1, sem1)
        for j in (1, 2, 3):
            h0.wait()
            h0 = pltpu.async_copy(tables[j].at[idx0.at[pl.ds(j * GP, GP)]],
                                  rows0, sem0, add=True)
            h1.wait()
            h1 = pltpu.async_copy(tables[j].at[idx1.at[pl.ds(j * GP, GP)]],
                                  rows1, sem1, add=True)
        h0.wait()
        s0 = pltpu.async_copy(rows0, out.at[c0], sem0)
        h1.wait()
        s1 = pltpu.async_copy(rows1, out.at[c1], sem1)
        s0.wait()
        s1.wait()


@functools.partial(
    pl.kernel,
    out_type=jax.ShapeDtypeStruct((2 * NACC, 128), jnp.float32),
    mesh=_MESH,
    scratch_types=[
        pltpu.VMEM((2, PCH), jnp.int32),
        pltpu.VMEM((PCH, 128), jnp.float32),
        pltpu.VMEM((PCH, 128), jnp.float32),
        pltpu.VMEM_SHARED((NACC, 128), jnp.float32),
    ],
)
def _sc_scatter(raw_e, raw_o, rse_h, rso_h, zero_h, out,
                idx2, rows_e, rows_o, acc):
    cid = lax.axis_index("c")
    sid = lax.axis_index("s")
    wid = sid * 2 + cid
    pltpu.sync_copy(zero_h.at[pl.ds(sid * RPT, RPT)],
                    acc.at[pl.ds(sid * RPT, RPT)])
    plsc.subcore_barrier()

    def body(t, carry):
        cb = wid * CPW + t
        pltpu.sync_copy(rse_h.at[cb], idx2.at[0])
        pltpu.sync_copy(rso_h.at[cb], idx2.at[1])
        pltpu.sync_copy(raw_e.at[cb], rows_e)
        pltpu.sync_copy(raw_o.at[cb], rows_o)
        pltpu.sync_copy(rows_e, acc.at[idx2.at[0]], add=True)
        pltpu.sync_copy(rows_o, acc.at[idx2.at[1]], add=True)
        return carry

    lax.fori_loop(0, CPW, body, 0)
    plsc.subcore_barrier()
    pltpu.sync_copy(acc.at[pl.ds(sid * RPT, RPT)],
                    out.at[pl.ds(cid * NACC + sid * RPT, RPT)])


def _full(shape):
    return pl.BlockSpec(shape, lambda *_: tuple(0 for _ in shape))


def _ln(x, scale, bias):
    mu = jnp.mean(x, axis=-1, keepdims=True)
    var = jnp.mean((x - mu) ** 2, axis=-1, keepdims=True)
    return (x - mu) / jnp.sqrt(var + LN_EPS) * scale + bias


def _enc_body(x_ref, w_ref, b_ref, wsa_ref, wsb_ref, wra_ref, wrb_ref,
              nodes_ref, tsa_ref, tsb_ref, tra_ref, trb_ref):
    n = jnp.dot(x_ref[...], w_ref[...]) + b_ref[...]
    nodes_ref[...] = n
    tsa_ref[...] = jnp.dot(n, wsa_ref[...])
    tsb_ref[...] = jnp.dot(n, wsb_ref[...])
    tra_ref[...] = jnp.dot(n, wra_ref[...])
    trb_ref[...] = jnp.dot(n, wrb_ref[...])


def _encode(x, w, b, wsa, wsb, wra, wrb):
    t128 = jax.ShapeDtypeStruct((N, 128), jnp.float32)
    return pl.pallas_call(
        _enc_body,
        out_shape=(jax.ShapeDtypeStruct((N, DH), jnp.float32),
                   t128, t128, t128, t128),
    )(x, w, b, wsa, wsb, wra, wrb)


def _edge_body(has_prev, want_ln, *refs):
    if has_prev:
        (pre_ref, eln_ref, w1e_ref, b1_ref, w2_ref, b2_ref,
         w3_ref, b3_ref, lns_ref, lnb_ref, *outs) = refs
    else:
        (pre_ref, b1_ref, w2_ref, b2_ref,
         w3_ref, b3_ref, lns_ref, lnb_ref, *outs) = refs
    x = pre_ref[...] + b1_ref[...]
    if has_prev:
        x = x + jnp.dot(eln_ref[...], w1e_ref[...])
    h = jnp.dot(jax.nn.gelu(x), w2_ref[...]) + b2_ref[...]
    h = jnp.dot(jax.nn.gelu(h), w3_ref[...]) + b3_ref[...]
    z = jnp.zeros_like(h[:, :DH])
    outs[0][...] = jnp.concatenate([h[:, :DH], z], axis=1)
    outs[1][...] = jnp.concatenate([h[:, DH:], z], axis=1)
    if want_ln:
        lne = _ln(h[:, :DH], lns_ref[...], lnb_ref[...])
        lno = _ln(h[:, DH:], lns_ref[...], lnb_ref[...])
        outs[2][...] = jnp.concatenate([lne, lno], axis=1)


def _edge_step(has_prev, want_ln, pre2, eln2, w1e_bd, b1_bd, w2_bd, b2_bd,
               w3_bd, b3_bd, lns, lnb):
    blk128 = lambda: pl.BlockSpec((EBLK, 128), lambda i: (i, 0))
    in_specs = [blk128()]
    args = [pre2]
    if has_prev:
        in_specs += [blk128(), _full((128, 128))]
        args += [eln2, w1e_bd]
    in_specs += [_full((1, 128)), _full((128, 128)), _full((1, 128)),
                 _full((128, 128)), _full((1, 128)),
                 _full((1, DH)), _full((1, DH))]
    args += [b1_bd, w2_bd, b2_bd, w3_bd, b3_bd, lns, lnb]
    out_specs = [blk128(), blk128()]
    out_shape = [jax.ShapeDtypeStruct((PAIRS, 128), jnp.float32),
                 jax.ShapeDtypeStruct((PAIRS, 128), jnp.float32)]
    if want_ln:
        out_specs.append(blk128())
        out_shape.append(jax.ShapeDtypeStruct((PAIRS, 128), jnp.float32))
    res = pl.pallas_call(
        functools.partial(_edge_body, has_prev, want_ln),
        grid=(PAIRS // EBLK,),
        in_specs=in_specs,
        out_specs=tuple(out_specs),
        out_shape=tuple(out_shape),
    )(*args)
    return res if want_ln else (res[0], res[1], None)


def _node_body(last, *refs):
    (nodes_ref, rec_ref, w1a_ref, w1b_ref, b1_ref, w2_ref, b2_ref,
     w3_ref, b3_ref, lns_ref, lnb_ref, *rest) = refs
    rec = (rec_ref[pl.ds(0, N), :] + rec_ref[pl.ds(NACC, N), :])[:, :DH]
    pre = (jnp.dot(nodes_ref[...], w1a_ref[...])
           + jnp.dot(rec, w1b_ref[...]) + b1_ref[...])
    h = jnp.dot(jax.nn.gelu(pre), w2_ref[...]) + b2_ref[...]
    h = jnp.dot(jax.nn.gelu(h), w3_ref[...]) + b3_ref[...]
    nodes_ln = _ln(h, lns_ref[...], lnb_ref[...])
    if not last:
        (wsa_ref, wsb_ref, wra_ref, wrb_ref,
         nodes_out, tsa_out, tsb_out, tra_out, trb_out) = rest
        nodes_out[...] = nodes_ln
        tsa_out[...] = jnp.dot(nodes_ln, wsa_ref[...])
        tsb_out[...] = jnp.dot(nodes_ln, wsb_ref[...])
        tra_out[...] = jnp.dot(nodes_ln, wra_ref[...])
        trb_out[...] = jnp.dot(nodes_ln, wrb_ref[...])
    else:
        (rw1_ref, rb1_ref, rw2_ref, rb2_ref, rw3_ref, rb3_ref,
         rw4_ref, rb4_ref, out_ref) = rest
        m = jnp.mean(nodes_ln, axis=0, keepdims=True)
        h = jax.nn.gelu(jnp.dot(m, rw1_ref[...]) + rb1_ref[...])
        h = jax.nn.gelu(jnp.dot(h, rw2_ref[...]) + rb2_ref[...])
        h = jax.nn.gelu(jnp.dot(h, rw3_ref[...]) + rb3_ref[...])
        out_ref[...] = jnp.dot(h, rw4_ref[...]) + rb4_ref[...]


def _node_step(nargs, wsa, wsb, wra, wrb):
    t128 = jax.ShapeDtypeStruct((N, 128), jnp.float32)
    return pl.pallas_call(
        functools.partial(_node_body, False),
        out_shape=(jax.ShapeDtypeStruct((N, DH), jnp.float32),
                   t128, t128, t128, t128),
    )(*nargs, wsa, wsb, wra, wrb)


def _node_readout(nargs, ro_args):
    return pl.pallas_call(
        functools.partial(_node_body, True),
        out_shape=jax.ShapeDtypeStruct((1, 1), jnp.float32),
    )(*nargs, *ro_args)


def _bd(w):
    z = jnp.zeros_like(w)
    return jnp.concatenate(
        [jnp.concatenate([w, z], axis=1), jnp.concatenate([z, w], axis=1)],
        axis=0)


def _padA(w):
    return jnp.concatenate([w, jnp.zeros_like(w)], axis=1)


def _padB(w):
    return jnp.concatenate([jnp.zeros_like(w), w], axis=1)


def kernel(x, edge_index, params):
    senders = edge_index[0].astype(jnp.int32)
    receivers = edge_index[1].astype(jnp.int32)
    pad = EP - E
    s_pad = jnp.concatenate([senders, jnp.zeros((pad,), jnp.int32)])
    r_pad = jnp.concatenate([receivers, jnp.zeros((pad,), jnp.int32)])
    rs_pad = jnp.concatenate([receivers, jnp.full((pad,), N, jnp.int32)])
    idx4 = jnp.concatenate([s_pad[0::2].reshape(NCHG, GP),
                            s_pad[1::2].reshape(NCHG, GP),
                            r_pad[0::2].reshape(NCHG, GP),
                            r_pad[1::2].reshape(NCHG, GP)], axis=1)
    rse = rs_pad[0::2].reshape(NCHP, PCH)
    rso = rs_pad[1::2].reshape(NCHP, PCH)
    zero_acc = jnp.zeros((NACC, 128), jnp.float32)

    def row(v):
        return v.reshape(1, -1)

    def row2(v):
        return jnp.concatenate([v, v]).reshape(1, -1)

    steps = params["steps"]
    w1_0 = steps[0]["edge_mlp"][0]["W"]
    nodes, tsa, tsb, tra, trb = _encode(
        x, params["enc"]["W"], row(params["enc"]["b"]),
        _padA(w1_0[:DH]), _padB(w1_0[:DH]),
        _padA(w1_0[DH:2 * DH]), _padB(w1_0[DH:2 * DH]))

    eln2 = None
    for t, sp in enumerate(steps):
        em, nm = sp["edge_mlp"], sp["node_mlp"]
        pre2 = _sc_gather(tsa, tsb, tra, trb, idx4)
        pre2 = pre2.reshape(PAIRS, 128)
        has_prev = t > 0
        want_ln = t < 2
        raw_e, raw_o, eln2 = _edge_step(
            has_prev, want_ln, pre2, eln2,
            _bd(em[0]["W"][:DH]) if has_prev else None,
            row2(em[0]["b"]), _bd(em[1]["W"]), row2(em[1]["b"]),
            _bd(em[2]["W"]), row2(em[2]["b"]),
            row(sp["ln_edges"]["scale"]), row(sp["ln_edges"]["bias"]))
        rec = _sc_scatter(raw_e.reshape(NCHP, PCH, 128),
                          raw_o.reshape(NCHP, PCH, 128), rse, rso, zero_acc)
        nw1 = nm[0]["W"]
        nargs = (nodes, rec, nw1[:DH], nw1[DH:], row(nm[0]["b"]),
                 nm[1]["W"], row(nm[1]["b"]), nm[2]["W"], row(nm[2]["b"]),
                 row(sp["ln_nodes"]["scale"]), row(sp["ln_nodes"]["bias"]))
        if t < 2:
            wn = steps[t + 1]["edge_mlp"][0]["W"]
            nodes, tsa, tsb, tra, trb = _node_step(
                nargs, _padA(wn[DH:2 * DH]), _padB(wn[DH:2 * DH]),
                _padA(wn[2 * DH:]), _padB(wn[2 * DH:]))
        else:
            ro_p = params["readout"]
            ro_args = (ro_p[0]["W"], row(ro_p[0]["b"]),
                       ro_p[1]["W"], row(ro_p[1]["b"]),
                       ro_p[2]["W"], row(ro_p[2]["b"]),
                       ro_p[3]["W"], row(ro_p[3]["b"]))
            out = _node_readout(nargs, ro_args)
    return out[0]
